# baseline (speedup 1.0000x reference)
import jax
import jax.numpy as jnp
from jax.experimental import pallas as pl
from jax.experimental.pallas import tpu as pltpu

N_TOK = 32768
IN_DIM = 128
FEAT_DIM = 128
ATTN_DIM = 64
NUM_CLASSES = 2
N_BAGS = 16

BLK = 8192
NB = N_TOK // BLK


def _tanh_f32(x):
    x = jnp.clip(x, -7.90531110763549805, 7.90531110763549805)
    x2 = x * x
    a = jnp.float32(-2.76076847742355e-16)
    a = a * x2 + jnp.float32(2.00018790482477e-13)
    a = a * x2 + jnp.float32(-8.60467152213735e-11)
    a = a * x2 + jnp.float32(5.12229709037114e-08)
    a = a * x2 + jnp.float32(1.48572235717979e-05)
    a = a * x2 + jnp.float32(6.37261928875436e-04)
    a = a * x2 + jnp.float32(4.89352455891786e-03)
    num = x * a
    b = jnp.float32(1.19825839466702e-06)
    b = b * x2 + jnp.float32(1.18534705686654e-04)
    b = b * x2 + jnp.float32(2.26843463243900e-03)
    b = b * x2 + jnp.float32(4.89352518554385e-03)
    return num / b


def _mil_kernel(x_ref, sizes_ref, W1_ref, b1_ref, Wa1_ref, ba1_ref,
                Wa2_ref, ba2_ref, Wc_ref, bc_ref, out_ref,
                num_ref, den_ref):
    i = pl.program_id(0)
    h = jnp.maximum(
        jnp.dot(x_ref[...], W1_ref[...],
                preferred_element_type=jnp.float32,
                precision=jax.lax.Precision.HIGHEST)
        + b1_ref[...], 0.0)
    t = _tanh_f32(
        jnp.dot(h, Wa1_ref[...],
                preferred_element_type=jnp.float32,
                precision=jax.lax.Precision.HIGHEST)
        + ba1_ref[...])
    s = (jnp.dot(t, Wa2_ref[...], preferred_element_type=jnp.float32,
                 precision=jax.lax.Precision.HIGHEST)
         + ba2_ref[...])
    c = jnp.sum(jnp.abs(Wa2_ref[...])) + jnp.abs(ba2_ref[0, 0])
    w = jnp.exp(s - c)

    sizes = sizes_ref[...]
    tri_r = jax.lax.broadcasted_iota(jnp.int32, (N_BAGS, N_BAGS), 0)
    tri_c = jax.lax.broadcasted_iota(jnp.int32, (N_BAGS, N_BAGS), 1)
    lower = (tri_r >= tri_c).astype(jnp.float32)
    ends_f = jnp.dot(lower, sizes,
                     preferred_element_type=jnp.float32)
    ends = ends_f.astype(jnp.int32)
    starts = (ends_f - sizes).astype(jnp.int32)
    lane = (jax.lax.broadcasted_iota(jnp.int32, (N_BAGS, BLK), 1)
            + i * BLK)
    member = ((lane >= starts) & (lane < ends)).astype(jnp.float32)
    num_p = jnp.dot(member, h * w,
                    preferred_element_type=jnp.float32,
                    precision=jax.lax.Precision.HIGHEST)
    den_p = jnp.dot(member, w,
                    preferred_element_type=jnp.float32,
                    precision=jax.lax.Precision.HIGHEST)

    first = (i == 0)
    num_ref[...] = jnp.where(first, num_p, num_ref[...] + num_p)
    den_ref[...] = jnp.where(first, den_p, den_ref[...] + den_p)

    @pl.when(i == NB - 1)
    def _finalize():
        emb = num_ref[...] / den_ref[...]
        out_ref[...] = (
            jnp.dot(emb, Wc_ref[...], preferred_element_type=jnp.float32)
            + bc_ref[...])


@jax.jit
def kernel(features, bag_sizes, W1, b1, Wa1, ba1, Wa2, ba2, Wc, bc):
    sizes_col = bag_sizes.astype(jnp.float32).reshape(N_BAGS, 1)
    return pl.pallas_call(
        _mil_kernel,
        grid=(NB,),
        in_specs=[
            pl.BlockSpec((BLK, IN_DIM), lambda i: (i, 0)),
            pl.BlockSpec((N_BAGS, 1), lambda i: (0, 0)),
            pl.BlockSpec((IN_DIM, FEAT_DIM), lambda i: (0, 0)),
            pl.BlockSpec((1, FEAT_DIM), lambda i: (0, 0)),
            pl.BlockSpec((FEAT_DIM, ATTN_DIM), lambda i: (0, 0)),
            pl.BlockSpec((1, ATTN_DIM), lambda i: (0, 0)),
            pl.BlockSpec((ATTN_DIM, 1), lambda i: (0, 0)),
            pl.BlockSpec((1, 1), lambda i: (0, 0)),
            pl.BlockSpec((FEAT_DIM, NUM_CLASSES), lambda i: (0, 0)),
            pl.BlockSpec((1, NUM_CLASSES), lambda i: (0, 0)),
        ],
        out_specs=pl.BlockSpec((N_BAGS, NUM_CLASSES), lambda i: (0, 0)),
        scratch_shapes=[
            pltpu.VMEM((N_BAGS, FEAT_DIM), jnp.float32),
            pltpu.VMEM((N_BAGS, 1), jnp.float32),
        ],
        out_shape=jax.ShapeDtypeStruct((N_BAGS, NUM_CLASSES), jnp.float32),
    )(features, sizes_col, W1, b1.reshape(1, -1), Wa1, ba1.reshape(1, -1),
      Wa2, ba2.reshape(1, -1), Wc, bc.reshape(1, -1))
